# Initial kernel scaffold; baseline (speedup 1.0000x reference)
import jax
import jax.numpy as jnp
from jax.experimental import pallas as pl


def kernel(feat, edge_index, in_norm, W1, b1, W2, b2):
    raise NotImplementedError("write your pallas kernel here")


# speedup vs baseline: 4.3925x; 4.3925x over previous
import functools

import jax
import jax.numpy as jnp
from jax import lax
from jax.experimental import pallas as pl
from jax.experimental.pallas import tpu as pltpu
from jax.experimental.pallas import tpu_sc as plsc

NC = 2
NS = 16
NW = NC * NS
K = 128


def _sc_aggregate(feat, src, dst, zeros, *, n, d, ew):
    n_acc = zeros.shape[0] * NS
    rows_z = zeros.shape[0]
    rows_out = 1000
    n_tiles_out = n // rows_out
    ch = ew // K

    mesh = plsc.VectorSubcoreMesh(core_axis_name="c", subcore_axis_name="s")

    @functools.partial(
        pl.kernel,
        out_type=jax.ShapeDtypeStruct((NC * n, d), jnp.float32),
        mesh=mesh,
        scratch_types=[
            pltpu.VMEM_SHARED((n_acc, d), jnp.float32),
            pltpu.VMEM((K,), jnp.int32),
            pltpu.VMEM((K,), jnp.int32),
            pltpu.VMEM((K, d), jnp.float32),
            pltpu.SemaphoreType.DMA,
        ],
    )
    def sc_kernel(feat_hbm, src_hbm, dst_hbm, zero_hbm, out_hbm,
                  acc, src_v, dst_v, rows_v, sem):
        c = lax.axis_index("c")
        s = lax.axis_index("s")
        wid = c * NS + s

        pltpu.sync_copy(zero_hbm, acc.at[pl.ds(s * rows_z, rows_z)])
        plsc.subcore_barrier()

        base = wid * ew

        def chunk(j, _):
            off = base + j * K
            pltpu.sync_copy(src_hbm.at[pl.ds(off, K)], src_v)
            pltpu.sync_copy(dst_hbm.at[pl.ds(off, K)], dst_v)
            pltpu.async_copy(feat_hbm.at[src_v], rows_v, sem).wait()
            pltpu.sync_copy(rows_v, acc.at[dst_v], add=True)
            return ()

        lax.fori_loop(0, ch, chunk, (), unroll=False)

        plsc.subcore_barrier()

        @pl.when(s < n_tiles_out)
        def _copy_out():
            pltpu.sync_copy(acc.at[pl.ds(s * rows_out, rows_out)],
                            out_hbm.at[pl.ds(c * n + s * rows_out, rows_out)])

    return sc_kernel(feat, src, dst, zeros)


def _tc_linear(feat, hp, norm, w1, w2, b1, b2, *, n, d, blk):
    nb = n // blk

    def body(feat_ref, h0_ref, h1_ref, norm_ref, w1_ref, w2_ref,
             b1_ref, b2_ref, out_ref):
        ah = (h0_ref[...] + h1_ref[...]) / norm_ref[...]
        dn = (((1,), (1,)), ((), ()))
        out_ref[...] = (
            lax.dot_general(feat_ref[...], w1_ref[...], dn,
                            preferred_element_type=jnp.float32)
            + lax.dot_general(ah, w2_ref[...], dn,
                              preferred_element_type=jnp.float32)
            + b1_ref[...] + b2_ref[...])

    return pl.pallas_call(
        body,
        grid=(nb,),
        in_specs=[
            pl.BlockSpec((blk, d), lambda i: (i, 0)),
            pl.BlockSpec((blk, d), lambda i: (i, 0)),
            pl.BlockSpec((blk, d), lambda i: (i + nb, 0)),
            pl.BlockSpec((blk, 1), lambda i: (i, 0)),
            pl.BlockSpec((d, d), lambda i: (0, 0)),
            pl.BlockSpec((d, d), lambda i: (0, 0)),
            pl.BlockSpec((1, d), lambda i: (0, 0)),
            pl.BlockSpec((1, d), lambda i: (0, 0)),
        ],
        out_specs=pl.BlockSpec((blk, d), lambda i: (i, 0)),
        out_shape=jax.ShapeDtypeStruct((n, d), jnp.float32),
    )(feat, hp, hp, norm, w1, w2, b1, b2)


def kernel(feat, edge_index, in_norm, W1, b1, W2, b2):
    n, d = feat.shape
    e = edge_index.shape[1]

    ew = -(-e // (NW * K)) * K
    pad = NW * ew - e
    src = jnp.concatenate([edge_index[0],
                           jnp.zeros((pad,), jnp.int32)])
    dst = jnp.concatenate([edge_index[1],
                           jnp.full((pad,), n, jnp.int32)])

    rows_z = -(-(n + 1) // (NS * 8)) * 8
    zeros = jnp.zeros((rows_z, d), jnp.float32)

    hp = _sc_aggregate(feat, src, dst, zeros, n=n, d=d, ew=ew)
    return _tc_linear(feat, hp, in_norm[:, None], W1, W2,
                      b1[None, :], b2[None, :], n=n, d=d, blk=1000)
